# baseline (speedup 1.0000x reference)
import functools

import jax
import jax.numpy as jnp
from jax import lax
from jax.experimental import pallas as pl
from jax.experimental.pallas import tpu as pltpu
from jax.experimental.pallas import tpu_sc as plsc

V = 100000
D = 64
K = 512

_NC = 2
_NS = 16
_NW = _NC * _NS

_B = 4096 * 50
_BPW = _B // _NW
_CHUNK = 128
_NCHUNK = _BPW // _CHUNK
_SEQ = 50
_PAD_SEQ = 56

_MM_BLOCK = 2000


def _table_matmul_body(code_ref, w_ref, out_ref):
    out_ref[:, :D] = jnp.dot(code_ref[...], w_ref[...],
                             preferred_element_type=jnp.float32)
    out_ref[:, D:] = jnp.zeros((_MM_BLOCK, D), jnp.float32)


def _build_table(code2d, w):
    grid = V // _MM_BLOCK
    return pl.pallas_call(
        _table_matmul_body,
        grid=(grid,),
        in_specs=[
            pl.BlockSpec((_MM_BLOCK, K), lambda i: (i, 0)),
            pl.BlockSpec((K, D), lambda i: (0, 0)),
        ],
        out_specs=pl.BlockSpec((_MM_BLOCK, 2 * D), lambda i: (i, 0)),
        out_shape=jax.ShapeDtypeStruct((V, 2 * D), jnp.float32),
    )(code2d, w)


def _gather_body(table_hbm, idx_hbm, dst_hbm, out_hbm, idx_v, dst_v, rows_v,
                 sem0, sem1, osem):
    wid = lax.axis_index("s") * _NC + lax.axis_index("c")
    pltpu.sync_copy(idx_hbm.at[pl.ds(wid * _NCHUNK, _NCHUNK)], idx_v)
    pltpu.sync_copy(dst_hbm.at[pl.ds(wid * _NCHUNK, _NCHUNK)], dst_v)

    sems = (sem0, sem1)
    copies = [None, None]
    copies[0] = pltpu.async_copy(table_hbm.at[idx_v.at[0]], rows_v.at[0],
                                 sems[0])
    for j in range(1, _NCHUNK):
        bsel = j % 2
        copies[bsel] = pltpu.async_copy(table_hbm.at[idx_v.at[j]],
                                        rows_v.at[bsel], sems[bsel])
        prev = (j - 1) % 2
        copies[prev].wait()
        pltpu.async_copy(rows_v.at[prev], out_hbm.at[dst_v.at[j - 1]],
                         osem).wait()
    last = (_NCHUNK - 1) % 2
    copies[last].wait()
    pltpu.async_copy(rows_v.at[last], out_hbm.at[dst_v.at[_NCHUNK - 1]],
                     osem).wait()


def _gather(table, idx2d, dst2d):
    mesh = plsc.VectorSubcoreMesh(core_axis_name="c", subcore_axis_name="s")
    return pl.kernel(
        _gather_body,
        out_type=jax.ShapeDtypeStruct((4096 * _PAD_SEQ, 2 * D), jnp.float32),
        mesh=mesh,
        scratch_types=[
            pltpu.VMEM((_NCHUNK, _CHUNK), jnp.int32),
            pltpu.VMEM((_NCHUNK, _CHUNK), jnp.int32),
            pltpu.VMEM((2, _CHUNK, 2 * D), jnp.float32),
            pltpu.SemaphoreType.DMA,
            pltpu.SemaphoreType.DMA,
            pltpu.SemaphoreType.DMA,
        ],
        compiler_params=pltpu.CompilerParams(use_tc_tiling_on_sc=False),
    )(table, idx2d, dst2d)


_C_BATCH = 64


def _final_copy_body(in_ref, out_ref):
    for b in range(_C_BATCH):
        out_ref[b] = in_ref[pl.ds(b * _PAD_SEQ, _SEQ), :D]


def _final_copy(buf):
    grid = 4096 // _C_BATCH
    return pl.pallas_call(
        _final_copy_body,
        grid=(grid,),
        in_specs=[pl.BlockSpec((_C_BATCH * _PAD_SEQ, 128), lambda i: (i, 0))],
        out_specs=pl.BlockSpec((_C_BATCH, _SEQ, D), lambda i: (i, 0, 0)),
        out_shape=jax.ShapeDtypeStruct((4096, _SEQ, D), jnp.float32),
    )(buf)


@jax.jit
def kernel(input, code, codebook):
    code2d = code.reshape(V, K)
    w = codebook.reshape(K, D)
    table = _build_table(code2d, w)
    idx2d = input.reshape(1600, 128).astype(jnp.int32)
    p = lax.iota(jnp.int32, _B)
    dst2d = (_PAD_SEQ * (p // _SEQ) + p % _SEQ).reshape(1600, 128)
    buf = _gather(table, idx2d, dst2d)
    return _final_copy(buf)


# speedup vs baseline: 1.1920x; 1.1920x over previous
import functools

import jax
import jax.numpy as jnp
from jax import lax
from jax.experimental import pallas as pl
from jax.experimental.pallas import tpu as pltpu
from jax.experimental.pallas import tpu_sc as plsc

V = 100000
HALF_V = V // 2
D = 64
K = 512

_NC = 2
_NS = 16
_NW = _NC * _NS

_B = 4096 * 50
_BPW = _B // _NW
_CHUNK = 128
_NCHUNK = _BPW // _CHUNK
_SEQ = 50

_MM_BLOCK = 2000


def _table_matmul_body(code_lo_ref, code_hi_ref, w_ref, out_ref):
    out_ref[:, :D] = jnp.dot(code_lo_ref[...], w_ref[...],
                             preferred_element_type=jnp.float32)
    out_ref[:, D:] = jnp.dot(code_hi_ref[...], w_ref[...],
                             preferred_element_type=jnp.float32)


def _build_table(code2d, w):
    grid = HALF_V // _MM_BLOCK
    return pl.pallas_call(
        _table_matmul_body,
        grid=(grid,),
        in_specs=[
            pl.BlockSpec((_MM_BLOCK, K), lambda i: (i, 0)),
            pl.BlockSpec((_MM_BLOCK, K), lambda i, g=grid: (i + g, 0)),
            pl.BlockSpec((K, D), lambda i: (0, 0)),
        ],
        out_specs=pl.BlockSpec((_MM_BLOCK, 2 * D), lambda i: (i, 0)),
        out_shape=jax.ShapeDtypeStruct((HALF_V, 2 * D), jnp.float32),
    )(code2d, code2d, w)


def _gather_body(table_hbm, gidx_hbm, out_hbm, gidx_v, rows_v,
                 sem0, sem1, osem):
    wid = lax.axis_index("s") * _NC + lax.axis_index("c")
    base = wid * _BPW
    pltpu.sync_copy(gidx_hbm.at[pl.ds(wid * _NCHUNK, _NCHUNK)], gidx_v)

    sems = (sem0, sem1)
    copies = [None, None]
    copies[0] = pltpu.async_copy(table_hbm.at[gidx_v.at[0]], rows_v.at[0],
                                 sems[0])
    for j in range(1, _NCHUNK):
        bsel = j % 2
        copies[bsel] = pltpu.async_copy(table_hbm.at[gidx_v.at[j]],
                                        rows_v.at[bsel], sems[bsel])
        prev = (j - 1) % 2
        copies[prev].wait()
        pltpu.async_copy(rows_v.at[prev],
                         out_hbm.at[pl.ds(base + (j - 1) * _CHUNK, _CHUNK)],
                         osem).wait()
    last = (_NCHUNK - 1) % 2
    copies[last].wait()
    pltpu.async_copy(rows_v.at[last],
                     out_hbm.at[pl.ds(base + (_NCHUNK - 1) * _CHUNK, _CHUNK)],
                     osem).wait()


def _gather(table64, gidx2d):
    mesh = plsc.VectorSubcoreMesh(core_axis_name="c", subcore_axis_name="s")
    return pl.kernel(
        _gather_body,
        out_type=jax.ShapeDtypeStruct((_B, D), jnp.float32),
        mesh=mesh,
        scratch_types=[
            pltpu.VMEM((_NCHUNK, _CHUNK), jnp.int32),
            pltpu.VMEM((2, _CHUNK, D), jnp.float32),
            pltpu.SemaphoreType.DMA,
            pltpu.SemaphoreType.DMA,
            pltpu.SemaphoreType.DMA,
        ],
        compiler_params=pltpu.CompilerParams(use_tc_tiling_on_sc=False),
    )(table64, gidx2d)


@jax.jit
def kernel(input, code, codebook):
    code2d = code.reshape(V, K)
    w = codebook.reshape(K, D)
    table = _build_table(code2d, w)
    table64 = table.reshape(V, D)
    v = input.reshape(-1).astype(jnp.int32)
    hi = (v >= HALF_V).astype(jnp.int32)
    gidx2d = (2 * v - (2 * HALF_V - 1) * hi).reshape(1600, 128)
    return _gather(table64, gidx2d).reshape(4096, _SEQ, D)
